# Initial kernel scaffold; baseline (speedup 1.0000x reference)
import functools

import jax
import jax.numpy as jnp
from jax import lax
from jax.experimental import pallas as pl
from jax.experimental.pallas import tpu as pltpu
from jax.experimental.pallas import tpu_sc as plsc

VOCAB = 1000000
EMBED = 64
BATCH = 16384
NUM_NEG = 20

NC, NS, L = 2, 16, 16
NW = NC * NS
B_PER_W = BATCH // NW
CHUNK = 64
NSTEPS = B_PER_W // CHUNK
NEG_ROWS = CHUNK * NUM_NEG
NIDX_SPLIT = NEG_ROWS // 128


def _sc_scores(center, context, neg2d, w_center, w_context):
    mesh = plsc.VectorSubcoreMesh(core_axis_name="c", subcore_axis_name="s")

    @functools.partial(
        pl.kernel,
        out_type=(
            jax.ShapeDtypeStruct((BATCH,), jnp.float32),
            jax.ShapeDtypeStruct((BATCH * NUM_NEG,), jnp.float32),
        ),
        mesh=mesh,
        scratch_types=[
            pltpu.VMEM((CHUNK,), jnp.int32),
            pltpu.VMEM((CHUNK,), jnp.int32),
            pltpu.VMEM((NIDX_SPLIT, 128), jnp.int32),
            pltpu.VMEM((CHUNK, EMBED), jnp.float32),
            pltpu.VMEM((CHUNK, EMBED), jnp.float32),
            pltpu.VMEM((NEG_ROWS, EMBED), jnp.float32),
            pltpu.VMEM((CHUNK,), jnp.float32),
            pltpu.VMEM((NEG_ROWS,), jnp.float32),
            pltpu.SemaphoreType.DMA,
        ],
    )
    def scores_kernel(center_h, context_h, neg_h, wc_h, wx_h,
                      pos_h, neg_out_h,
                      idx_c, idx_x, idx_n, rows_c, rows_x, rows_n,
                      pos_v, neg_v, sem):
        wid = lax.axis_index("s") * NC + lax.axis_index("c")
        base = wid * B_PER_W
        for step in range(NSTEPS):
            gb = base + step * CHUNK
            pltpu.sync_copy(center_h.at[pl.ds(gb, CHUNK)], idx_c)
            pltpu.sync_copy(context_h.at[pl.ds(gb, CHUNK)], idx_x)
            pltpu.sync_copy(
                neg_h.at[pl.ds(gb * NUM_NEG // 128, NIDX_SPLIT)], idx_n)
            descs = [
                pltpu.async_copy(wc_h.at[idx_c], rows_c, sem),
                pltpu.async_copy(wx_h.at[idx_x], rows_x, sem),
            ]
            for j in range(NIDX_SPLIT):
                descs.append(pltpu.async_copy(
                    wx_h.at[idx_n.at[j]],
                    rows_n.at[pl.ds(j * 128, 128)], sem))
            for d in descs:
                d.wait()
            for g in range(CHUNK // L):
                rows16 = lax.iota(jnp.int32, L) + g * L
                row_k = [rows16 * NUM_NEG + k for k in range(NUM_NEG)]

                def dim_body(dd, accs, rows16=rows16, row_k=row_k):
                    col = jnp.full((L,), dd, jnp.int32)
                    v = plsc.load_gather(rows_c, [rows16, col])
                    up = plsc.load_gather(rows_x, [rows16, col])
                    new = [accs[0] + v * up]
                    for k in range(NUM_NEG):
                        un = plsc.load_gather(rows_n, [row_k[k], col])
                        new.append(accs[k + 1] + v * un)
                    return tuple(new)

                accs = lax.fori_loop(
                    0, EMBED, dim_body,
                    tuple(jnp.zeros((L,), jnp.float32)
                          for _ in range(NUM_NEG + 1)))
                pos_v[pl.ds(g * L, L)] = accs[0]
                for k in range(NUM_NEG):
                    plsc.store_scatter(neg_v, [row_k[k]], accs[k + 1])
            pltpu.sync_copy(pos_v, pos_h.at[pl.ds(gb, CHUNK)])
            pltpu.sync_copy(neg_v, neg_out_h.at[pl.ds(gb * NUM_NEG, NEG_ROWS)])

    return scores_kernel(center, context, neg2d, w_center, w_context)


def _loss_kernel(pos_ref, neg_ref, out_ref):
    def log_sigmoid(x):
        return jnp.minimum(x, 0.0) - jnp.log1p(jnp.exp(-jnp.abs(x)))

    total = (jnp.sum(log_sigmoid(pos_ref[...]))
             + jnp.sum(log_sigmoid(-neg_ref[...])))
    out_ref[0, 0] = -total / BATCH


def kernel(center, context, negatives, W_center, W_context):
    center = center.astype(jnp.int32)
    context = context.astype(jnp.int32)
    neg2d = negatives.astype(jnp.int32).reshape(BATCH * NUM_NEG // 128, 128)
    pos, neg = _sc_scores(center, context, neg2d, W_center, W_context)
    loss = pl.pallas_call(
        _loss_kernel,
        out_shape=jax.ShapeDtypeStruct((1, 1), jnp.float32),
        in_specs=[
            pl.BlockSpec(memory_space=pltpu.VMEM),
            pl.BlockSpec(memory_space=pltpu.VMEM),
        ],
        out_specs=pl.BlockSpec(memory_space=pltpu.SMEM),
    )(pos.reshape(BATCH // 128, 128), neg.reshape(BATCH * NUM_NEG // 128, 128))
    return loss[0, 0]


# speedup vs baseline: 3.9750x; 3.9750x over previous
import functools

import jax
import jax.numpy as jnp
from jax import lax
from jax.experimental import pallas as pl
from jax.experimental.pallas import tpu as pltpu
from jax.experimental.pallas import tpu_sc as plsc

VOCAB = 1000000
EMBED = 64
BATCH = 16384
NUM_NEG = 20

NC, NS, L = 2, 16, 16
NW = NC * NS
B_PER_W = BATCH // NW
CHUNK = 64
NSTEPS = B_PER_W // CHUNK
NEG_ROWS = CHUNK * NUM_NEG
NIDX_SPLIT = NEG_ROWS // 128


def _sc_scores(center, context, neg2d, w_center, w_context):
    mesh = plsc.VectorSubcoreMesh(core_axis_name="c", subcore_axis_name="s")

    @functools.partial(
        pl.kernel,
        out_type=(
            jax.ShapeDtypeStruct((BATCH,), jnp.float32),
            jax.ShapeDtypeStruct((BATCH * NUM_NEG,), jnp.float32),
        ),
        mesh=mesh,
        scratch_types=[
            pltpu.VMEM((CHUNK,), jnp.int32),
            pltpu.VMEM((CHUNK,), jnp.int32),
            pltpu.VMEM((NEG_ROWS,), jnp.int32),
            pltpu.VMEM((CHUNK, EMBED), jnp.float32),
            pltpu.VMEM((CHUNK, EMBED), jnp.float32),
            pltpu.VMEM((NEG_ROWS, EMBED), jnp.float32),
            pltpu.VMEM((CHUNK,), jnp.float32),
            pltpu.VMEM((NEG_ROWS,), jnp.float32),
            pltpu.SemaphoreType.DMA,
        ],
        compiler_params=pltpu.CompilerParams(
            needs_layout_passes=False, use_tc_tiling_on_sc=False),
    )
    def scores_kernel(center_h, context_h, neg_h, wc_h, wx_h,
                      pos_h, neg_out_h,
                      idx_c, idx_x, idx_n, rows_c, rows_x, rows_n,
                      pos_v, neg_v, sem):
        wid = lax.axis_index("s") * NC + lax.axis_index("c")
        base = wid * B_PER_W
        for step in range(NSTEPS):
            gb = base + step * CHUNK
            pltpu.sync_copy(center_h.at[pl.ds(gb, CHUNK)], idx_c)
            pltpu.sync_copy(context_h.at[pl.ds(gb, CHUNK)], idx_x)
            pltpu.sync_copy(neg_h.at[pl.ds(gb * NUM_NEG, NEG_ROWS)], idx_n)
            descs = [
                pltpu.async_copy(wc_h.at[idx_c], rows_c, sem),
                pltpu.async_copy(wx_h.at[idx_x], rows_x, sem),
            ]
            for j in range(NIDX_SPLIT):
                descs.append(pltpu.async_copy(
                    wx_h.at[idx_n.at[pl.ds(j * 128, 128)]],
                    rows_n.at[pl.ds(j * 128, 128)], sem))
            for d in descs:
                d.wait()
            for g in range(CHUNK // L):
                rows16 = lax.iota(jnp.int32, L) + g * L
                row_k = [rows16 * NUM_NEG + k for k in range(NUM_NEG)]

                def dim_body(dd, accs, rows16=rows16, row_k=row_k):
                    col = jnp.full((L,), dd, jnp.int32)
                    v = plsc.load_gather(rows_c, [rows16, col])
                    up = plsc.load_gather(rows_x, [rows16, col])
                    new = [accs[0] + v * up]
                    for k in range(NUM_NEG):
                        un = plsc.load_gather(rows_n, [row_k[k], col])
                        new.append(accs[k + 1] + v * un)
                    return tuple(new)

                accs = lax.fori_loop(
                    0, EMBED, dim_body,
                    tuple(jnp.zeros((L,), jnp.float32)
                          for _ in range(NUM_NEG + 1)))
                pos_v[pl.ds(g * L, L)] = accs[0]
                for k in range(NUM_NEG):
                    plsc.store_scatter(neg_v, [row_k[k]], accs[k + 1])
            pltpu.sync_copy(pos_v, pos_h.at[pl.ds(gb, CHUNK)])
            pltpu.sync_copy(neg_v, neg_out_h.at[pl.ds(gb * NUM_NEG, NEG_ROWS)])

    return scores_kernel(center, context, neg2d, w_center, w_context)


def _loss_kernel(pos_ref, neg_ref, out_ref):
    def log_sigmoid(x):
        return jnp.minimum(x, 0.0) - jnp.log1p(jnp.exp(-jnp.abs(x)))

    total = (jnp.sum(log_sigmoid(pos_ref[...]))
             + jnp.sum(log_sigmoid(-neg_ref[...])))
    out_ref[0, 0] = -total / BATCH


def kernel(center, context, negatives, W_center, W_context):
    center = center.astype(jnp.int32)
    context = context.astype(jnp.int32)
    neg_flat = negatives.astype(jnp.int32).reshape(BATCH * NUM_NEG)
    pos, neg = _sc_scores(center, context, neg_flat, W_center, W_context)
    loss = pl.pallas_call(
        _loss_kernel,
        out_shape=jax.ShapeDtypeStruct((1, 1), jnp.float32),
        in_specs=[
            pl.BlockSpec(memory_space=pltpu.VMEM),
            pl.BlockSpec(memory_space=pltpu.VMEM),
        ],
        out_specs=pl.BlockSpec(memory_space=pltpu.SMEM),
    )(pos.reshape(BATCH // 128, 128), neg.reshape(BATCH * NUM_NEG // 128, 128))
    return loss[0, 0]
